# Initial kernel scaffold; baseline (speedup 1.0000x reference)
import jax
import jax.numpy as jnp
from jax.experimental import pallas as pl


def kernel(x, edge_index, batch, target, Wa1, ba1, Wa2, ba2, Wg, att_src, att_dst, bg, W1, b1, W2, b2, Wc1, bc1, Wc2, bc2, emb, Ws, bs, Wxt, bxt, Wf, bf, Wo, bo):
    raise NotImplementedError("write your pallas kernel here")


# speedup vs baseline: 1.0403x; 1.0403x over previous
import jax
import jax.numpy as jnp
from jax.experimental import pallas as pl

NEG = -3e38


def _dot(a, b, dims):
    return jax.lax.dot_general(a, b, (dims, ((), ())),
                               preferred_element_type=jnp.float32)


def _ids(base_i, nb):
    it = jax.lax.broadcasted_iota(jnp.int32, (1, nb), 1)
    return (base_i * nb + it).astype(jnp.float32)


def _node_prep_k(x_ref, wa1, ba1, wa2, ba2, wg, as_ref, ad_ref,
                 h_ref, asrc_ref, adst_ref):
    x = x_ref[...]
    a = 0.5 * (jax.nn.sigmoid(_dot(x, wa1[...], ((1,), (0,))) + ba1[...]) +
               jax.nn.sigmoid(_dot(x, wa2[...], ((1,), (0,))) + ba2[...]))
    t = x * a
    s = jnp.sum(t, axis=1, keepdims=True) + 1e-12
    x1 = t / s
    x1 = jnp.where(jnp.isnan(x1) | jnp.isinf(x1), 0.0, x1)
    h = _dot(x1, wg[...], ((1,), (0,)))
    h_ref[...] = h
    asrc_ref[...] = _dot(h, as_ref[...], ((1,), (0,)))
    adst_ref[...] = _dot(h, ad_ref[...], ((1,), (0,)))


def _edge_e_k(src_ref, dst_ref, asrc_ref, adst_ref, e_ref, *, Nb, nN):
    j = pl.program_id(1)

    @pl.when(j == 0)
    def _():
        e_ref[...] = jnp.zeros_like(e_ref)

    ids = _ids(j, Nb)
    oh_s = (src_ref[0] == ids).astype(jnp.float32)
    oh_d = (dst_ref[0] == ids).astype(jnp.float32)
    acc = (e_ref[...] + _dot(oh_s, asrc_ref[...], ((1,), (0,))) +
           _dot(oh_d, adst_ref[...], ((1,), (0,))))

    @pl.when(j == nN - 1)
    def _():
        e_ref[...] = jnp.where(acc >= 0, acc, 0.2 * acc)

    @pl.when(j < nN - 1)
    def _():
        e_ref[...] = acc


def _seg_max_k(dst_ref, e_ref, m_ref, *, Nb, H):
    i = pl.program_id(0)
    j = pl.program_id(1)

    @pl.when(j == 0)
    def _():
        m_ref[...] = jnp.full_like(m_ref, NEG)

    ids = _ids(i, Nb)
    oh = dst_ref[0] == ids
    e = e_ref[...]
    for k in range(H):
        ek = e[:, k:k + 1]
        mk = jnp.max(jnp.where(oh, ek, NEG), axis=0, keepdims=True)
        m_ref[0, k:k + 1, :] = jnp.maximum(m_ref[0, k:k + 1, :], mk)


def _edge_ex_k(dst_ref, m_ref, e_ref, ex_ref, *, Nb, nN):
    j = pl.program_id(1)

    @pl.when(j == 0)
    def _():
        ex_ref[...] = jnp.zeros_like(ex_ref)

    ids = _ids(j, Nb)
    oh = (dst_ref[0] == ids).astype(jnp.float32)
    acc = ex_ref[...] + _dot(oh, m_ref[0], ((1,), (1,)))

    @pl.when(j == nN - 1)
    def _():
        ex_ref[...] = jnp.exp(e_ref[...] - acc)

    @pl.when(j < nN - 1)
    def _():
        ex_ref[...] = acc


def _seg_sum_k(dst_ref, ex_ref, ss_ref, *, Nb):
    i = pl.program_id(0)
    j = pl.program_id(1)

    @pl.when(j == 0)
    def _():
        ss_ref[...] = jnp.zeros_like(ss_ref)

    ids = _ids(i, Nb)
    oh = (dst_ref[0] == ids).astype(jnp.float32)
    ss_ref[...] += _dot(oh, ex_ref[...], ((0,), (0,)))


def _alpha_k(dst_ref, ss_ref, ex_ref, al_ref, *, Nb, nN):
    j = pl.program_id(1)

    @pl.when(j == 0)
    def _():
        al_ref[...] = jnp.zeros_like(al_ref)

    ids = _ids(j, Nb)
    oh = (dst_ref[0] == ids).astype(jnp.float32)
    acc = al_ref[...] + _dot(oh, ss_ref[...], ((1,), (0,)))

    @pl.when(j == nN - 1)
    def _():
        al_ref[...] = ex_ref[...] / (acc + 1e-16)

    @pl.when(j < nN - 1)
    def _():
        al_ref[...] = acc


def _xg_k(src_ref, dst_ref, al_ref, h_ref, rep_ref, xg_ref, *, Nb, nN, D):
    i = pl.program_id(0)

    @pl.when(i == 0)
    def _():
        xg_ref[...] = jnp.zeros_like(xg_ref)

    src = src_ref[0]
    dst = dst_ref[0]
    Eb = src.shape[0]

    def gbody(jj, hs):
        oh = (src == _ids(jj, Nb)).astype(jnp.float32)
        return hs + _dot(oh, h_ref[pl.ds(jj * Nb, Nb), :], ((1,), (0,)))

    hs = jax.lax.fori_loop(0, nN, gbody, jnp.zeros((Eb, D), jnp.float32))
    msg = hs * _dot(al_ref[...], rep_ref[...], ((1,), (0,)))

    def sbody(jj, carry):
        oh = (dst == _ids(jj, Nb)).astype(jnp.float32)
        xg_ref[pl.ds(jj * Nb, Nb), :] += _dot(oh, msg, ((0,), (0,)))
        return carry

    jax.lax.fori_loop(0, nN, sbody, 0)


def _agg_k(src_ref, dst_ref, xg_ref, bg_ref, agg_ref, *, Nb, nN, D):
    i = pl.program_id(0)

    @pl.when(i == 0)
    def _():
        agg_ref[...] = jnp.zeros_like(agg_ref)

    src = src_ref[0]
    dst = dst_ref[0]
    Eb = src.shape[0]

    def gbody(jj, hs):
        oh = (src == _ids(jj, Nb)).astype(jnp.float32)
        return hs + _dot(oh, xg_ref[pl.ds(jj * Nb, Nb), :], ((1,), (0,)))

    hs = jax.lax.fori_loop(0, nN, gbody, jnp.zeros((Eb, D), jnp.float32))
    xgs = hs + bg_ref[...]

    def sbody(jj, carry):
        oh = (dst == _ids(jj, Nb)).astype(jnp.float32)
        agg_ref[pl.ds(jj * Nb, Nb), :] += _dot(oh, xgs, ((0,), (0,)))
        return carry

    jax.lax.fori_loop(0, nN, sbody, 0)


def _z_k(xg_ref, agg_ref, bg_ref, w1, b1, w2, b2, z_ref):
    t = xg_ref[...] + bg_ref[...] + agg_ref[...]
    t = jnp.maximum(_dot(t, w1[...], ((1,), (0,))) + b1[...], 0.0)
    t = _dot(t, w2[...], ((1,), (0,))) + b2[...]
    z_ref[...] = jnp.maximum(t, 0.0)


def _pool_k(batch_ref, z_ref, gmax_ref, gsum_ref, cnt_ref, *, B):
    i = pl.program_id(0)

    @pl.when(i == 0)
    def _():
        gmax_ref[...] = jnp.full_like(gmax_ref, NEG)
        gsum_ref[...] = jnp.zeros_like(gsum_ref)
        cnt_ref[...] = jnp.zeros_like(cnt_ref)

    bv = batch_ref[0]
    z = z_ref[...]
    idsB = jax.lax.broadcasted_iota(jnp.int32, (1, B), 1).astype(jnp.float32)
    ohB = (bv == idsB).astype(jnp.float32)
    gsum_ref[...] += _dot(ohB, z, ((0,), (0,)))
    cnt_ref[...] += _dot(ohB, jnp.ones((bv.shape[0], 8), jnp.float32),
                         ((0,), (0,)))

    def body(b, carry):
        mask = bv == b.astype(jnp.float32)
        mk = jnp.max(jnp.where(mask, z, NEG), axis=0, keepdims=True)
        gmax_ref[pl.ds(b, 1), :] = jnp.maximum(gmax_ref[pl.ds(b, 1), :], mk)
        return carry

    jax.lax.fori_loop(0, B, body, 0)


def _prot_k(t_ref, pt_ref, mt_ref, bs_ref, wxt_ref, bxt_ref, xt_ref, *, V, G):
    tv = t_ref[0]
    ids = jax.lax.broadcasted_iota(jnp.int32, (1, V), 1).astype(jnp.float32)
    oh = (tv == ids).astype(jnp.float32)
    c = jnp.maximum(_dot(oh, pt_ref[...], ((1,), (0,))) + bs_ref[...], 0.0)
    cm = _dot(mt_ref[...], c, ((1,), (0,)))
    s = jnp.sum(cm, axis=1, keepdims=True) + 1e-12
    v = cm / s
    v = jnp.where(jnp.isnan(v) | jnp.isinf(v), 0.0, v)

    acc = jnp.zeros((1, 128), jnp.float32)
    for g in range(G):
        acc = acc + _dot(v[g:g + 1, :], wxt_ref[g * 16:(g + 1) * 16, :],
                         ((1,), (0,)))
    xt_ref[0] = acc + bxt_ref[...]


def _final_k(gmax_ref, gsum_ref, cnt_ref, xt_ref, wc1a, wc1b, bc1, wc2, bc2,
             wfa, wfb, bf, wo, bo, out_ref):
    gmax = gmax_ref[...]
    gmax = jnp.where(gmax <= NEG * 0.5, 0.0, gmax)
    cnt = jnp.maximum(cnt_ref[:, 0:1], 1.0)
    gmean = gsum_ref[...] / cnt
    xp = jnp.maximum(_dot(gmax, wc1a[...], ((1,), (0,))) +
                     _dot(gmean, wc1b[...], ((1,), (0,))) + bc1[...], 0.0)
    xp = jnp.maximum(_dot(xp, wc2[...], ((1,), (0,))) + bc2[...], 0.0)
    xc = jnp.maximum(_dot(xp, wfa[...], ((1,), (0,))) +
                     _dot(xt_ref[...], wfb[...], ((1,), (0,))) + bf[...], 0.0)
    out_ref[...] = jax.nn.sigmoid(_dot(xc, wo[...], ((1,), (0,))) + bo[...])


def _full(shape):
    return pl.BlockSpec(shape, lambda *a: tuple(0 for _ in shape))


def kernel(x, edge_index, batch, target, Wa1, ba1, Wa2, ba2, Wg, att_src,
           att_dst, bg, W1, b1, W2, b2, Wc1, bc1, Wc2, bc2, emb, Ws, bs,
           Wxt, bxt, Wf, bf, Wo, bo):
    f32 = jnp.float32
    N = x.shape[0]
    E = edge_index.shape[1]
    B, L = target.shape
    H, F = att_src.shape
    D = H * F
    V = emb.shape[0]
    G = D // H
    R = L // G
    Nb = 1000 if N % 1000 == 0 else N
    Eb = 1600 if E % 1600 == 0 else E
    nN = N // Nb
    nE = E // Eb

    eye = jnp.eye(H, dtype=f32)
    As = jnp.einsum('kf,kj->kfj', att_src.astype(f32), eye).reshape(D, H)
    Ad = jnp.einsum('kf,kj->kfj', att_dst.astype(f32), eye).reshape(D, H)
    Rep = jnp.repeat(eye, F, axis=1)
    PT = emb @ Ws.T
    MT = jnp.repeat(jnp.eye(G, dtype=f32), R, axis=1) / R
    Wxt2 = Wxt.T.reshape(16, G, 128).transpose(1, 0, 2).reshape(G * 16, 128)
    Wc1T = Wc1.T
    Wc1a, Wc1b = Wc1T[:D], Wc1T[D:]
    Wc2T = Wc2.T
    WfA, WfB = Wf[:128], Wf[128:]
    bg2 = bg.reshape(1, D)
    rs = lambda v: v.reshape(1, -1)

    srcf = edge_index[0].astype(f32).reshape(nE, Eb, 1)
    dstf = edge_index[1].astype(f32).reshape(nE, Eb, 1)
    batchf = batch.astype(f32).reshape(nN, Nb, 1)
    targf = target.astype(f32).reshape(B, L, 1)

    nspec = pl.BlockSpec((Nb, x.shape[1]), lambda i: (i, 0))

    h, asrc, adst = pl.pallas_call(
        _node_prep_k,
        grid=(nN,),
        in_specs=[nspec, _full(Wa1.shape), _full((1, 35)), _full(Wa2.shape),
                  _full((1, 35)), _full(Wg.shape), _full(As.shape),
                  _full(Ad.shape)],
        out_specs=[pl.BlockSpec((Nb, D), lambda i: (i, 0)),
                   pl.BlockSpec((Nb, H), lambda i: (i, 0)),
                   pl.BlockSpec((Nb, H), lambda i: (i, 0))],
        out_shape=[jax.ShapeDtypeStruct((N, D), f32),
                   jax.ShapeDtypeStruct((N, H), f32),
                   jax.ShapeDtypeStruct((N, H), f32)],
    )(x, Wa1, rs(ba1), Wa2, rs(ba2), Wg, As, Ad)

    espec = pl.BlockSpec((1, Eb, 1), lambda i, j: (i, 0, 0))
    eblk = pl.BlockSpec((Eb, H), lambda i, j: (i, 0))
    nblk = pl.BlockSpec((Nb, H), lambda i, j: (j, 0))

    import functools
    e = pl.pallas_call(
        functools.partial(_edge_e_k, Nb=Nb, nN=nN),
        grid=(nE, nN),
        in_specs=[espec, espec, nblk, nblk],
        out_specs=eblk,
        out_shape=jax.ShapeDtypeStruct((E, H), f32),
    )(srcf, dstf, asrc, adst)

    espec2 = pl.BlockSpec((1, Eb, 1), lambda i, j: (j, 0, 0))
    eblk2 = pl.BlockSpec((Eb, H), lambda i, j: (j, 0))
    mT = pl.pallas_call(
        functools.partial(_seg_max_k, Nb=Nb, H=H),
        grid=(nN, nE),
        in_specs=[espec2, eblk2],
        out_specs=pl.BlockSpec((1, H, Nb), lambda i, j: (i, 0, 0)),
        out_shape=jax.ShapeDtypeStruct((nN, H, Nb), f32),
    )(dstf, e)

    mblk = pl.BlockSpec((1, H, Nb), lambda i, j: (j, 0, 0))
    ex = pl.pallas_call(
        functools.partial(_edge_ex_k, Nb=Nb, nN=nN),
        grid=(nE, nN),
        in_specs=[espec, mblk, eblk],
        out_specs=eblk,
        out_shape=jax.ShapeDtypeStruct((E, H), f32),
    )(dstf, mT, e)

    ssum = pl.pallas_call(
        functools.partial(_seg_sum_k, Nb=Nb),
        grid=(nN, nE),
        in_specs=[espec2, eblk2],
        out_specs=pl.BlockSpec((Nb, H), lambda i, j: (i, 0)),
        out_shape=jax.ShapeDtypeStruct((N, H), f32),
    )(dstf, ex)

    alpha = pl.pallas_call(
        functools.partial(_alpha_k, Nb=Nb, nN=nN),
        grid=(nE, nN),
        in_specs=[espec, nblk, eblk],
        out_specs=eblk,
        out_shape=jax.ShapeDtypeStruct((E, H), f32),
    )(dstf, ssum, ex)

    espec1 = pl.BlockSpec((1, Eb, 1), lambda i: (i, 0, 0))
    eblk1 = pl.BlockSpec((Eb, H), lambda i: (i, 0))
    xg_raw = pl.pallas_call(
        functools.partial(_xg_k, Nb=Nb, nN=nN, D=D),
        grid=(nE,),
        in_specs=[espec1, espec1, eblk1,
                  pl.BlockSpec((N, D), lambda i: (0, 0)),
                  pl.BlockSpec((H, D), lambda i: (0, 0))],
        out_specs=pl.BlockSpec((N, D), lambda i: (0, 0)),
        out_shape=jax.ShapeDtypeStruct((N, D), f32),
    )(srcf, dstf, alpha, h, Rep)

    agg = pl.pallas_call(
        functools.partial(_agg_k, Nb=Nb, nN=nN, D=D),
        grid=(nE,),
        in_specs=[espec1, espec1,
                  pl.BlockSpec((N, D), lambda i: (0, 0)),
                  pl.BlockSpec((1, D), lambda i: (0, 0))],
        out_specs=pl.BlockSpec((N, D), lambda i: (0, 0)),
        out_shape=jax.ShapeDtypeStruct((N, D), f32),
    )(srcf, dstf, xg_raw, bg2)

    ndblk = pl.BlockSpec((Nb, D), lambda i: (i, 0))
    z = pl.pallas_call(
        _z_k,
        grid=(nN,),
        in_specs=[ndblk, ndblk, _full((1, D)), _full(W1.shape),
                  _full((1, D)), _full(W2.shape), _full((1, D))],
        out_specs=ndblk,
        out_shape=jax.ShapeDtypeStruct((N, D), f32),
    )(xg_raw, agg, bg2, W1, rs(b1), W2, rs(b2))

    gmax, gsum, cnt8 = pl.pallas_call(
        functools.partial(_pool_k, B=B),
        grid=(nN,),
        in_specs=[pl.BlockSpec((1, Nb, 1), lambda i: (i, 0, 0)), ndblk],
        out_specs=[pl.BlockSpec((B, D), lambda i: (0, 0)),
                   pl.BlockSpec((B, D), lambda i: (0, 0)),
                   pl.BlockSpec((B, 8), lambda i: (0, 0))],
        out_shape=[jax.ShapeDtypeStruct((B, D), f32),
                   jax.ShapeDtypeStruct((B, D), f32),
                   jax.ShapeDtypeStruct((B, 8), f32)],
    )(batchf, z)

    xt = pl.pallas_call(
        functools.partial(_prot_k, V=V, G=G),
        grid=(B,),
        in_specs=[pl.BlockSpec((1, L, 1), lambda i: (i, 0, 0)),
                  _full(PT.shape), _full(MT.shape), _full((1, 16)),
                  _full(Wxt2.shape), _full((1, 128))],
        out_specs=pl.BlockSpec((1, 1, 128), lambda i: (i, 0, 0)),
        out_shape=jax.ShapeDtypeStruct((B, 1, 128), f32),
    )(targf, PT, MT, rs(bs), Wxt2, rs(bxt))
    xt = xt.reshape(B, 128)

    out = pl.pallas_call(
        _final_k,
        in_specs=[_full((B, D)), _full((B, D)), _full((B, 8)),
                  _full((B, 128)), _full(Wc1a.shape), _full(Wc1b.shape),
                  _full((1, Wc1a.shape[1])), _full(Wc2T.shape),
                  _full((1, 128)), _full(WfA.shape), _full(WfB.shape),
                  _full((1, WfA.shape[1])), _full(Wo.shape), _full((1, 1))],
        out_specs=_full((B, 1)),
        out_shape=jax.ShapeDtypeStruct((B, 1), f32),
    )(gmax, gsum, cnt8, xt, Wc1a, Wc1b, rs(bc1), Wc2T, rs(bc2), WfA, WfB,
      rs(bf), Wo, rs(bo))

    return out, alpha
